# speedup vs baseline: 40.5344x; 40.5344x over previous
import jax
import jax.numpy as jnp
from jax.experimental import pallas as pl

_SHAPE = (64, 32768)
_N = _SHAPE[0] * _SHAPE[1]
_K = int(0.1 * _N)


def _select_mask_kernel(x_ref, out_ref):
    xf = x_ref[...]
    u = jax.lax.bitcast_convert_type(xf, jnp.int32) & jnp.int32(0x7FFFFFFF)

    def body(i, t):
        cand = t | (jnp.int32(1) << (jnp.int32(30) - i))
        cnt = jnp.sum((u >= cand).astype(jnp.int32))
        return jnp.where(cnt >= _K, cand, t)

    t = jax.lax.fori_loop(0, 31, body, jnp.int32(0))
    out_ref[...] = jnp.where(u >= t, xf, 0.0)


def kernel(x):
    return pl.pallas_call(
        _select_mask_kernel,
        out_shape=jax.ShapeDtypeStruct(_SHAPE, jnp.float32),
    )(x)
